# baseline (speedup 1.0000x reference)
import jax
import jax.numpy as jnp
from jax.experimental import pallas as pl


def _conv1d(x, w, b, pad):
    y = jax.lax.conv_general_dilated(
        x, w, window_strides=(1,), padding=[(pad, pad)],
        dimension_numbers=('NCH', 'OIH', 'NCH'))
    return y + b[None, :, None]


def _maxpool5(x):
    return jax.lax.reduce_window(x, -jnp.inf, jax.lax.max, (1, 1, 5), (1, 1, 5), 'VALID')


def _cls_kernel(pooled_ref, clW1_ref, clb1_ref, ln_g_ref, ln_b_ref,
                clW2_ref, clb2_ref, out_ref):
    m = jnp.dot(pooled_ref[...], clW1_ref[...],
                preferred_element_type=jnp.float32) + clb1_ref[...]
    mu = jnp.mean(m, axis=-1, keepdims=True)
    var = jnp.mean((m - mu) ** 2, axis=-1, keepdims=True)
    m = (m - mu) / jnp.sqrt(var + 1e-5) * ln_g_ref[...] + ln_b_ref[...]
    m = jnp.maximum(m, 0.0)
    y = jnp.dot(m, clW2_ref[...], preferred_element_type=jnp.float32) + clb2_ref[...]
    out_ref[...] = jax.nn.sigmoid(y)


def kernel(x, edge_index, edge_weight, batch, cw0, cb0, cw1, cb1, cw2, cb2,
           Wr0, br0, Wt0, Wr1, br1, Wt1, Wr2, br2, Wt2,
           clW1, clb1, ln_g, ln_b, clW2, clb2):
    n = x.shape[0]
    src, dst = edge_index[0], edge_index[1]

    z = x[:, None, :]
    z = jnp.maximum(_conv1d(z, cw0, cb0, 9 // 2), 0.0)
    z = _maxpool5(z)
    t = jnp.maximum(_conv1d(z, cw1, cb1, 2), 0.0)
    z = jnp.maximum(t + z, 0.0)
    z = jnp.maximum(_conv1d(z, cw2, cb2, 2), 0.0)
    z = _maxpool5(z)
    h = z.reshape(n, -1)

    def gconv(h_in, Wr, br, Wt):
        msg = h_in[src] * edge_weight[:, None]
        agg = jax.ops.segment_sum(msg, dst, num_segments=n)
        return agg @ Wr + br + h_in @ Wt

    h = jnp.maximum(gconv(h, Wr0, br0, Wt0), 0.0)
    t = jnp.maximum(gconv(h, Wr1, br1, Wt1), 0.0)
    h = jnp.maximum(t + h, 0.0)
    h = jnp.maximum(gconv(h, Wr2, br2, Wt2), 0.0)

    G = 64
    summed = jax.ops.segment_sum(h, batch, num_segments=G)
    cnt = jax.ops.segment_sum(jnp.ones((n, 1), dtype=h.dtype), batch, num_segments=G)
    pooled = summed / jnp.maximum(cnt, 1.0)

    out = pl.pallas_call(
        _cls_kernel,
        out_shape=jax.ShapeDtypeStruct((G, 3), jnp.float32),
    )(pooled, clW1, clb1[None, :], ln_g[None, :], ln_b[None, :],
      clW2, clb2[None, :])
    return out


# speedup vs baseline: 3.3248x; 3.3248x over previous
import functools

import jax
import jax.numpy as jnp
from jax import lax
from jax.experimental import pallas as pl
from jax.experimental.pallas import tpu as pltpu
from jax.experimental.pallas import tpu_sc as plsc

N = 50000
E = 800000
E2 = 819200
NCHUNK = E2 // 128
NSUB = 16
NCORE = 2
NCH_SUB = NCHUNK // NSUB
SB = 16
NSB = NCH_SUB // SB
RING = 4
NPAD = 50048
NROWS_SUB = NPAD // NSUB

BE = 400
BL = 1000
NBL = N // BL


def _make_sc_segsum(ng, w):
    npass = ng // NCORE
    mesh = plsc.VectorSubcoreMesh(core_axis_name="c", subcore_axis_name="s")

    @functools.partial(
        pl.kernel,
        out_type=jax.ShapeDtypeStruct((ng, NPAD, w), jnp.float32),
        mesh=mesh,
        scratch_types=[
            pltpu.VMEM_SHARED((NPAD, w), jnp.float32),
            pltpu.VMEM((RING, 128, w), jnp.float32),
            pltpu.VMEM((RING, 128), jnp.int32),
            pltpu.VMEM((2 * SB, 128), jnp.int32),
            pltpu.VMEM((2 * SB, 128), jnp.int32),
            pltpu.VMEM((2 * SB, 128), jnp.float32),
            pltpu.SemaphoreType.DMA,
            pltpu.SemaphoreType.DMA,
            pltpu.SemaphoreType.DMA,
        ],
        compiler_params=pltpu.CompilerParams(use_tc_tiling_on_sc=False),
    )
    def k(table, src2d, dst2d, ew2d, zeros, out,
          shared, rows, sidx, sibuf, dibuf, ewbuf, gsem, ssem, isem):
        core = lax.axis_index("c")
        s = lax.axis_index("s")

        def idx_pairs(b, sl):
            row0 = s * NCH_SUB + b * SB
            return [(hbm.at[pl.ds(row0, SB)], buf.at[pl.ds(sl * SB, SB)])
                    for hbm, buf in ((src2d, sibuf), (dst2d, dibuf),
                                     (ew2d, ewbuf))]

        for p in range(npass):
            cg = core * npass + p
            nz = s * NROWS_SUB
            pltpu.sync_copy(zeros.at[pl.ds(nz, NROWS_SUB)],
                            shared.at[pl.ds(nz, NROWS_SUB)])
            plsc.subcore_barrier()

            for sref, dref in idx_pairs(0, 0):
                pltpu.sync_copy(sref, dref)

            def sb_body(b, carry, cg=cg):
                sl = lax.rem(b, 2)
                nsl = 1 - sl
                have_next = b + 1 < NSB

                @pl.when(have_next)
                def _prefetch():
                    for sref, dref in idx_pairs(b + 1, nsl):
                        pltpu.async_copy(sref, dref, isem)

                def group_body(gi, c2, sl=sl, cg=cg):
                    gdescs = []
                    for r in range(RING):
                        rl = sl * SB + gi * RING + r
                        for k16 in range(8):
                            v = sibuf[rl, pl.ds(k16 * 16, 16)]
                            sidx[r, pl.ds(k16 * 16, 16)] = v * ng + cg
                        gdescs.append(pltpu.async_copy(
                            table.at[sidx.at[r]], rows.at[r], gsem))
                    for d in gdescs:
                        d.wait()
                    sdescs = []
                    for r in range(RING):
                        rl = sl * SB + gi * RING + r

                        def scale_body(g, c3, rl=rl, r=r):
                            wv = ewbuf[rl, pl.ds(g * 16, 16)]
                            for j in range(16):
                                e = g * 16 + j
                                for hh in range(w // 16):
                                    rows[r, e, pl.ds(hh * 16, 16)] = (
                                        rows[r, e, pl.ds(hh * 16, 16)] * wv[j])
                            return c3
                        lax.fori_loop(0, 8, scale_body, 0)
                        sdescs.append(pltpu.async_copy(
                            rows.at[r], shared.at[dibuf.at[rl]], ssem,
                            add=True))
                    for d in sdescs:
                        d.wait()
                    return c2
                lax.fori_loop(0, SB // RING, group_body, 0)

                @pl.when(have_next)
                def _drain_prefetch():
                    for sref, dref in idx_pairs(b + 1, nsl):
                        pltpu.make_async_copy(sref, dref, isem).wait()
                return carry
            lax.fori_loop(0, NSB, sb_body, 0)

            plsc.subcore_barrier()
            pltpu.sync_copy(shared.at[pl.ds(nz, NROWS_SUB)],
                            out.at[cg, pl.ds(nz, NROWS_SUB)])

    return k


@functools.cache
def _sc_segsum(ng, w):
    return _make_sc_segsum(ng, w)


def _SC16(*args):
    return _sc_segsum(2, 16)(*args)


def _SC32(*args):
    return _sc_segsum(4, 32)(*args)


def _enc_body(xp_ref, w0_ref, b0_ref, w1_ref, b1_ref, w2_ref, b2_ref, out_ref):
    xb = xp_ref[...]
    zp = None
    for r5 in range(5):
        zr = jnp.maximum(
            jnp.dot(xb, w0_ref[r5], preferred_element_type=jnp.float32)
            + b0_ref[...], 0.0)
        zp = zr if zp is None else jnp.maximum(zp, zr)

    zzero = jnp.zeros((BE, 128), jnp.float32)
    zpad = jnp.concatenate([zzero, zp, zzero], axis=1)
    t1 = jnp.concatenate(
        [jnp.maximum(
            jnp.dot(zpad[:, q * 64:(q + 5) * 64], w1_ref[...],
                    preferred_element_type=jnp.float32) + b1_ref[...], 0.0)
         for q in range(20)], axis=1)
    z2 = jnp.maximum(t1 + zp, 0.0)

    z2pad = jnp.concatenate([zzero, z2, zzero], axis=1)
    h_parts = []
    for u in range(4):
        hu = None
        for r5 in range(5):
            q = 5 * u + r5
            z3q = jnp.maximum(
                jnp.dot(z2pad[:, q * 64:(q + 5) * 64], w2_ref[...],
                        preferred_element_type=jnp.float32) + b2_ref[...],
                0.0)
            hu = z3q if hu is None else jnp.maximum(hu, z3q)
        h_parts.append(hu)
    out_ref[...] = jnp.concatenate(h_parts, axis=1)


def _layer_body(agg_ref, h_ref, wr_ref, wt_ref, br_ref, out_ref, *, ng, skip):
    ag = jnp.concatenate([agg_ref[g] for g in range(ng)], axis=-1)
    acc = (jnp.dot(ag, wr_ref[...], preferred_element_type=jnp.float32)
           + jnp.dot(h_ref[...], wt_ref[...],
                     preferred_element_type=jnp.float32)
           + br_ref[...])
    a = jnp.maximum(acc, 0.0)
    if skip:
        a = jnp.maximum(a + h_ref[...], 0.0)
    out_ref[...] = a


def _final_body(agg_ref, h_ref, batch_ref, wr_ref, wt_ref, br_ref,
                clw1_ref, clb1_ref, lng_ref, lnb_ref, clw2_ref, clb2_ref,
                out_ref, sums, cnts):
    i = pl.program_id(0)

    @pl.when(i == 0)
    def _init():
        sums[...] = jnp.zeros_like(sums)
        cnts[...] = jnp.zeros_like(cnts)
        out_ref[...] = jnp.zeros_like(out_ref)

    ag = jnp.concatenate([agg_ref[g] for g in range(4)], axis=-1)
    acc = (jnp.dot(ag, wr_ref[...], preferred_element_type=jnp.float32)
           + jnp.dot(h_ref[...], wt_ref[...],
                     preferred_element_type=jnp.float32)
           + br_ref[...])
    h3 = jnp.maximum(acc, 0.0)

    bv = batch_ref[0, 0, :]
    oh = (bv[:, None] == lax.broadcasted_iota(jnp.int32, (BL, 64), 1)
          ).astype(jnp.float32)
    sums[...] += lax.dot_general(oh, h3, (((0,), (0,)), ((), ())),
                                 preferred_element_type=jnp.float32)
    cnts[...] += jnp.sum(oh, axis=0)[:, None]

    @pl.when(i == NBL - 1)
    def _cls():
        pooled = sums[...] / jnp.maximum(cnts[...], 1.0)
        m = (jnp.dot(pooled, clw1_ref[...],
                     preferred_element_type=jnp.float32) + clb1_ref[...])
        mu = jnp.mean(m, axis=-1, keepdims=True)
        var = jnp.mean((m - mu) ** 2, axis=-1, keepdims=True)
        m = (m - mu) * lax.rsqrt(var + 1e-5) * lng_ref[...] + lnb_ref[...]
        m = jnp.maximum(m, 0.0)
        y = (jnp.dot(m, clw2_ref[...],
                     preferred_element_type=jnp.float32) + clb2_ref[...])
        out_ref[...] = jax.nn.sigmoid(y)


def _enc_call(xp, w0, b0, w1m, b1, w2m, b2):
    return pl.pallas_call(
        _enc_body,
        grid=(N // BE,),
        in_specs=[
            pl.BlockSpec((BE, 108), lambda i: (i, 0)),
            pl.BlockSpec((5, 108, 1280), lambda i: (0, 0, 0)),
            pl.BlockSpec((1, 1280), lambda i: (0, 0)),
            pl.BlockSpec((320, 64), lambda i: (0, 0)),
            pl.BlockSpec((1, 64), lambda i: (0, 0)),
            pl.BlockSpec((320, 8), lambda i: (0, 0)),
            pl.BlockSpec((1, 8), lambda i: (0, 0)),
        ],
        out_specs=pl.BlockSpec((BE, 32), lambda i: (i, 0)),
        out_shape=jax.ShapeDtypeStruct((N, 32), jnp.float32),
    )(xp, w0, b0, w1m, b1, w2m, b2)


def _layer_call(agg, h, wr, wt, br, *, ng, w, skip):
    d_in = h.shape[1]
    return pl.pallas_call(
        functools.partial(_layer_body, ng=ng, skip=skip),
        grid=(NBL,),
        in_specs=[
            pl.BlockSpec((ng, BL, w), lambda i: (0, i, 0)),
            pl.BlockSpec((BL, d_in), lambda i: (i, 0)),
            pl.BlockSpec((ng * w, 128), lambda i: (0, 0)),
            pl.BlockSpec((d_in, 128), lambda i: (0, 0)),
            pl.BlockSpec((1, 128), lambda i: (0, 0)),
        ],
        out_specs=pl.BlockSpec((BL, 128), lambda i: (i, 0)),
        out_shape=jax.ShapeDtypeStruct((N, 128), jnp.float32),
    )(agg, h, wr, wt, br)


def _final_call(agg, h, batch3, wr, wt, br, clw1, clb1, lng, lnb, clw2, clb2):
    return pl.pallas_call(
        _final_body,
        grid=(NBL,),
        in_specs=[
            pl.BlockSpec((4, BL, 32), lambda i: (0, i, 0)),
            pl.BlockSpec((BL, 128), lambda i: (i, 0)),
            pl.BlockSpec((1, 1, BL), lambda i: (i, 0, 0)),
            pl.BlockSpec((128, 128), lambda i: (0, 0)),
            pl.BlockSpec((128, 128), lambda i: (0, 0)),
            pl.BlockSpec((1, 128), lambda i: (0, 0)),
            pl.BlockSpec((128, 64), lambda i: (0, 0)),
            pl.BlockSpec((1, 64), lambda i: (0, 0)),
            pl.BlockSpec((1, 64), lambda i: (0, 0)),
            pl.BlockSpec((1, 64), lambda i: (0, 0)),
            pl.BlockSpec((64, 3), lambda i: (0, 0)),
            pl.BlockSpec((1, 3), lambda i: (0, 0)),
        ],
        out_specs=pl.BlockSpec((64, 3), lambda i: (0, 0)),
        out_shape=jax.ShapeDtypeStruct((64, 3), jnp.float32),
        scratch_shapes=[
            pltpu.VMEM((64, 128), jnp.float32),
            pltpu.VMEM((64, 128), jnp.float32),
        ],
    )(agg, h, batch3, wr, wt, br, clw1, clb1, lng, lnb, clw2, clb2)


def kernel(x, edge_index, edge_weight, batch, cw0, cb0, cw1, cb1, cw2, cb2,
           Wr0, br0, Wt0, Wr1, br1, Wt1, Wr2, br2, Wt2,
           clW1, clb1, ln_g, ln_b, clW2, clb2):
    src, dst = edge_index[0], edge_index[1]
    pad = E2 - E
    src2d = jnp.concatenate([src, jnp.zeros((pad,), src.dtype)]).reshape(
        NCHUNK, 128)
    dst2d = jnp.concatenate([dst, jnp.zeros((pad,), dst.dtype)]).reshape(
        NCHUNK, 128)
    ew2d = jnp.concatenate(
        [edge_weight, jnp.zeros((pad,), edge_weight.dtype)]).reshape(
        NCHUNK, 128)
    zeros16 = jnp.zeros((NPAD, 16), jnp.float32)
    zeros32 = jnp.zeros((NPAD, 32), jnp.float32)

    xp = jnp.pad(x, ((0, 0), (4, 4)))
    cw0m = cw0[:, 0, :]
    jj = jnp.arange(108)[:, None]
    ll = jnp.arange(100)[None, :]
    tt = jj - ll
    valid = (tt >= 0) & (tt < 9)
    w0full = jnp.where(valid[:, :, None], cw0m.T[jnp.clip(tt, 0, 8)], 0.0)
    w0 = jnp.stack([w0full[:, r::5, :].reshape(108, 1280) for r in range(5)])
    b0 = jnp.tile(cb0, 20)[None, :]
    w1m = cw1.transpose(2, 1, 0).reshape(320, 64)
    w2m = cw2.transpose(2, 1, 0).reshape(320, 8)

    h0 = _enc_call(xp, w0, b0, w1m, cb1[None, :], w2m, cb2[None, :])

    perm = jnp.array([c * 4 + q for q in range(4) for c in range(8)])
    wr0p = Wr0[perm]
    wt0p = Wt0[perm]

    agg0 = _SC16(h0.reshape(N * 2, 16), src2d, dst2d, ew2d, zeros16)
    h1 = _layer_call(agg0, h0, wr0p, wt0p, br0[None, :], ng=2, w=16,
                     skip=False)
    agg1 = _SC32(h1.reshape(N * 4, 32), src2d, dst2d, ew2d, zeros32)
    h2 = _layer_call(agg1, h1, Wr1, Wt1, br1[None, :], ng=4, w=32, skip=True)
    agg2 = _SC32(h2.reshape(N * 4, 32), src2d, dst2d, ew2d, zeros32)
    return _final_call(agg2, h2, batch.reshape(NBL, 1, BL), Wr2, Wt2,
                       br2[None, :], clW1, clb1[None, :], ln_g[None, :],
                       ln_b[None, :], clW2, clb2[None, :])
